# baseline (speedup 1.0000x reference)
import jax
import jax.numpy as jnp
from jax.experimental import pallas as pl

N = 100000
EPS = 1e-08


def _copy_kernel(x_ref, o_ref):
    o_ref[...] = x_ref[...]


def kernel(pos, phi, edge_index):
    ei = jnp.concatenate([edge_index, edge_index[::-1, :]], axis=1)
    row, col = ei[0], ei[1]
    dx = pos[col] - pos[row]
    dphi = phi[col] - phi[row]
    w = 1.0 / (jnp.linalg.norm(dx, axis=1) + EPS) ** 2
    outer = dx[:, :, None] * dx[:, None, :]
    A = jax.ops.segment_sum(w[:, None, None] * outer, row, num_segments=N)
    b = jax.ops.segment_sum(w[:, None] * dphi[:, None] * dx, row, num_segments=N)
    eye = jnp.eye(3, dtype=pos.dtype)[None, :, :]
    A_reg = A + EPS * eye
    grad_phi = jnp.linalg.solve(A_reg, b[:, :, None])[:, :, 0]
    flat = jnp.pad(grad_phi.reshape(-1), (0, 300032 - 300000)).reshape(2344, 128)
    out = pl.pallas_call(
        _copy_kernel,
        out_shape=jax.ShapeDtypeStruct(flat.shape, flat.dtype),
    )(flat)
    return out.reshape(-1)[:300000].reshape(100000, 3)


# speedup vs baseline: 493.6681x; 493.6681x over previous
import jax
import jax.numpy as jnp
from jax import lax
from jax.experimental import pallas as pl
from jax.experimental.pallas import tpu as pltpu
from jax.experimental.pallas import tpu_sc as plsc

N = 100000
E = 1600000
EPS = 1e-8

NC = 2
NS = 16
NW = NC * NS
LANES = 128
K = 8
CL = K * LANES

N_PAD = 100352
E_PAD = 1605632
ROWS_PER_W = E_PAD // NW // LANES
CHUNKS = ROWS_PER_W // K
NODES_PER_W = N_PAD // NW
NODES_PER_TILE = N_PAD // NS

_mesh = plsc.VectorSubcoreMesh(core_axis_name="c", subcore_axis_name="s")


def _accumulate_body(r_hbm, c_hbm, px_hbm, py_hbm, pz_hbm, ph_hbm, zeros_hbm, out_hbm, *refs):
    idx_r = refs[0:K]
    idx_c = refs[K:2 * K]
    rbuf = refs[2 * K:2 * K + 4]
    cbuf = refs[2 * K + 4:2 * K + 8]
    vals = refs[2 * K + 8:2 * K + 17]
    accs = refs[2 * K + 17:2 * K + 26]
    gsem, ssem = refs[2 * K + 26:2 * K + 28]

    cid = lax.axis_index("c")
    sid = lax.axis_index("s")
    wid = sid * NC + cid

    zsl = pl.ds(sid * NODES_PER_TILE, NODES_PER_TILE)
    for k in range(9):
        pltpu.sync_copy(zeros_hbm.at[zsl], accs[k].at[zsl])
    plsc.subcore_barrier()

    def chunk(i, _):
        base = wid * ROWS_PER_W + i * K
        cps = []
        for j in range(K):
            cps.append(pltpu.async_copy(
                r_hbm.at[pl.ds((base + j) * LANES, LANES)], idx_r[j], gsem))
            cps.append(pltpu.async_copy(
                c_hbm.at[pl.ds((base + j) * LANES, LANES)], idx_c[j], gsem))
        for cp in cps:
            cp.wait()

        planes = (px_hbm, py_hbm, pz_hbm, ph_hbm)
        cps = []
        for j in range(K):
            dsl = pl.ds(j * LANES, LANES)
            for comp in range(4):
                cps.append(pltpu.async_copy(
                    planes[comp].at[idx_r[j]], rbuf[comp].at[dsl], gsem))
                cps.append(pltpu.async_copy(
                    planes[comp].at[idx_c[j]], cbuf[comp].at[dsl], gsem))
        for cp in cps:
            cp.wait()

        def group(gg, _):
            s = pl.ds(gg * 16, 16)
            dx0 = cbuf[0][s] - rbuf[0][s]
            dx1 = cbuf[1][s] - rbuf[1][s]
            dx2 = cbuf[2][s] - rbuf[2][s]
            dphi = cbuf[3][s] - rbuf[3][s]
            r2 = dx0 * dx0 + dx1 * dx1 + dx2 * dx2
            w = 1.0 / (r2 + EPS * EPS)
            wdx0 = w * dx0
            wdx1 = w * dx1
            wdx2 = w * dx2
            wdphi = w * dphi
            vals[0][s] = wdx0 * dx0
            vals[1][s] = wdx0 * dx1
            vals[2][s] = wdx0 * dx2
            vals[3][s] = wdx1 * dx1
            vals[4][s] = wdx1 * dx2
            vals[5][s] = wdx2 * dx2
            vals[6][s] = wdphi * dx0
            vals[7][s] = wdphi * dx1
            vals[8][s] = wdphi * dx2
            return _

        lax.fori_loop(0, K * 8, group, None)

        cps = []
        for j in range(K):
            dsl = pl.ds(j * LANES, LANES)
            for k in range(9):
                cps.append(pltpu.async_copy(
                    vals[k].at[dsl], accs[k].at[idx_r[j]], ssem, add=True))
                cps.append(pltpu.async_copy(
                    vals[k].at[dsl], accs[k].at[idx_c[j]], ssem, add=True))
        for cp in cps:
            cp.wait()
        return _

    lax.fori_loop(0, CHUNKS, chunk, None)

    plsc.subcore_barrier()
    for k in range(9):
        pltpu.sync_copy(
            accs[k].at[zsl],
            out_hbm.at[pl.ds((cid * 9 + k) * N_PAD + sid * NODES_PER_TILE,
                             NODES_PER_TILE)])


def _solve_body(part_hbm, grad_hbm, *refs):
    p0 = refs[0:9]
    p1 = refs[9:18]
    gbuf = refs[18:21]

    cid = lax.axis_index("c")
    sid = lax.axis_index("s")
    wid = sid * NC + cid
    base = wid * NODES_PER_W

    for k in range(9):
        pltpu.sync_copy(part_hbm.at[pl.ds(k * N_PAD + base, NODES_PER_W)], p0[k])
        pltpu.sync_copy(part_hbm.at[pl.ds((9 + k) * N_PAD + base, NODES_PER_W)], p1[k])

    def group(g, _):
        s = pl.ds(g * 16, 16)

        def ld(k):
            return p0[k][s] + p1[k][s]

        a0 = ld(0) + EPS
        a1 = ld(1)
        a2 = ld(2)
        a3 = ld(3) + EPS
        a4 = ld(4)
        a5 = ld(5) + EPS
        b0 = ld(6)
        b1 = ld(7)
        b2 = ld(8)
        c00 = a3 * a5 - a4 * a4
        c01 = a2 * a4 - a1 * a5
        c02 = a1 * a4 - a3 * a2
        c11 = a0 * a5 - a2 * a2
        c12 = a1 * a2 - a0 * a4
        c22 = a0 * a3 - a1 * a1
        inv = 1.0 / (a0 * c00 + a1 * c01 + a2 * c02)
        gbuf[0][s] = (c00 * b0 + c01 * b1 + c02 * b2) * inv
        gbuf[1][s] = (c01 * b0 + c11 * b1 + c12 * b2) * inv
        gbuf[2][s] = (c02 * b0 + c12 * b1 + c22 * b2) * inv
        return _

    lax.fori_loop(0, NODES_PER_W // 16, group, None)
    for k in range(3):
        pltpu.sync_copy(gbuf[k], grad_hbm.at[pl.ds(k * N_PAD + base, NODES_PER_W)])


_accumulate = pl.kernel(
    _accumulate_body,
    mesh=_mesh,
    out_type=jax.ShapeDtypeStruct((NC * 9 * N_PAD,), jnp.float32),
    scratch_types=(
        [pltpu.VMEM((LANES,), jnp.int32) for _ in range(2 * K)]
        + [pltpu.VMEM((CL,), jnp.float32) for _ in range(8)]
        + [pltpu.VMEM((CL,), jnp.float32) for _ in range(9)]
        + [pltpu.VMEM_SHARED((N_PAD,), jnp.float32) for _ in range(9)]
        + [pltpu.SemaphoreType.DMA, pltpu.SemaphoreType.DMA]
    ),
)

_solve = pl.kernel(
    _solve_body,
    mesh=_mesh,
    out_type=jax.ShapeDtypeStruct((3 * N_PAD,), jnp.float32),
    scratch_types=(
        [pltpu.VMEM((NODES_PER_W,), jnp.float32) for _ in range(18)]
        + [pltpu.VMEM((NODES_PER_W,), jnp.float32) for _ in range(3)]
    ),
)


def kernel(pos, phi, edge_index):
    ei = edge_index.astype(jnp.int32)
    pad = jnp.zeros((E_PAD - E,), jnp.int32)
    r_flat = jnp.concatenate([ei[0], pad])
    c_flat = jnp.concatenate([ei[1], pad])
    zeros = jnp.zeros((N_PAD,), jnp.float32)
    partials = _accumulate(r_flat, c_flat,
                           pos[:, 0], pos[:, 1], pos[:, 2], phi, zeros)
    grad = _solve(partials)
    return grad.reshape(3, N_PAD)[:, :N].T


# speedup vs baseline: 588.2385x; 1.1916x over previous
import jax
import jax.numpy as jnp
from jax import lax
from jax.experimental import pallas as pl
from jax.experimental.pallas import tpu as pltpu
from jax.experimental.pallas import tpu_sc as plsc

N = 100000
E = 1600000
EPS = 1e-8

NC = 2
NS = 16
NW = NC * NS
LANES = 128
K = 8
CL = K * LANES

N_PAD = 100352
E_PAD = 1605632
ROWS_PER_W = E_PAD // NW // LANES
CHUNKS = ROWS_PER_W // K
NODES_PER_W = N_PAD // NW
NODES_PER_TILE = N_PAD // NS

_mesh = plsc.VectorSubcoreMesh(core_axis_name="c", subcore_axis_name="s")


def _accumulate_body(r_hbm, c_hbm, px_hbm, py_hbm, pz_hbm, ph_hbm, zeros_hbm, out_hbm, *refs):
    idx_r = refs[0:K]
    idx_c = refs[K:2 * K]
    rbuf = refs[2 * K:2 * K + 4]
    cbuf = refs[2 * K + 4:2 * K + 8]
    vals = refs[2 * K + 8:2 * K + 17]
    accs = refs[2 * K + 17:2 * K + 26]
    tbls = refs[2 * K + 26:2 * K + 30]
    gsem, ssem = refs[2 * K + 30:2 * K + 32]

    cid = lax.axis_index("c")
    sid = lax.axis_index("s")
    wid = sid * NC + cid

    zsl = pl.ds(sid * NODES_PER_TILE, NODES_PER_TILE)
    for k in range(9):
        pltpu.sync_copy(zeros_hbm.at[zsl], accs[k].at[zsl])
    planes_hbm = (px_hbm, py_hbm, pz_hbm, ph_hbm)
    for comp in range(4):
        pltpu.sync_copy(planes_hbm[comp].at[zsl], tbls[comp].at[zsl])
    plsc.subcore_barrier()

    def chunk(i, _):
        base = wid * ROWS_PER_W + i * K
        cps = []
        for j in range(K):
            cps.append(pltpu.async_copy(
                r_hbm.at[pl.ds((base + j) * LANES, LANES)], idx_r[j], gsem))
            cps.append(pltpu.async_copy(
                c_hbm.at[pl.ds((base + j) * LANES, LANES)], idx_c[j], gsem))
        for cp in cps:
            cp.wait()

        cps = []
        for j in range(K):
            dsl = pl.ds(j * LANES, LANES)
            for comp in range(4):
                cps.append(pltpu.async_copy(
                    tbls[comp].at[idx_r[j]], rbuf[comp].at[dsl], gsem))
                cps.append(pltpu.async_copy(
                    tbls[comp].at[idx_c[j]], cbuf[comp].at[dsl], gsem))
        for cp in cps:
            cp.wait()

        def group(gg, _):
            s = pl.ds(gg * 16, 16)
            dx0 = cbuf[0][s] - rbuf[0][s]
            dx1 = cbuf[1][s] - rbuf[1][s]
            dx2 = cbuf[2][s] - rbuf[2][s]
            dphi = cbuf[3][s] - rbuf[3][s]
            r2 = dx0 * dx0 + dx1 * dx1 + dx2 * dx2
            w = 1.0 / (r2 + EPS * EPS)
            wdx0 = w * dx0
            wdx1 = w * dx1
            wdx2 = w * dx2
            wdphi = w * dphi
            vals[0][s] = wdx0 * dx0
            vals[1][s] = wdx0 * dx1
            vals[2][s] = wdx0 * dx2
            vals[3][s] = wdx1 * dx1
            vals[4][s] = wdx1 * dx2
            vals[5][s] = wdx2 * dx2
            vals[6][s] = wdphi * dx0
            vals[7][s] = wdphi * dx1
            vals[8][s] = wdphi * dx2
            return _

        lax.fori_loop(0, K * 8, group, None)

        cps = []
        for j in range(K):
            dsl = pl.ds(j * LANES, LANES)
            for k in range(9):
                cps.append(pltpu.async_copy(
                    vals[k].at[dsl], accs[k].at[idx_r[j]], ssem, add=True))
                cps.append(pltpu.async_copy(
                    vals[k].at[dsl], accs[k].at[idx_c[j]], ssem, add=True))
        for cp in cps:
            cp.wait()
        return _

    lax.fori_loop(0, CHUNKS, chunk, None)

    plsc.subcore_barrier()
    for k in range(9):
        pltpu.sync_copy(
            accs[k].at[zsl],
            out_hbm.at[pl.ds((cid * 9 + k) * N_PAD + sid * NODES_PER_TILE,
                             NODES_PER_TILE)])


def _solve_body(part_hbm, grad_hbm, *refs):
    p0 = refs[0:9]
    p1 = refs[9:18]
    gbuf = refs[18:21]

    cid = lax.axis_index("c")
    sid = lax.axis_index("s")
    wid = sid * NC + cid
    base = wid * NODES_PER_W

    for k in range(9):
        pltpu.sync_copy(part_hbm.at[pl.ds(k * N_PAD + base, NODES_PER_W)], p0[k])
        pltpu.sync_copy(part_hbm.at[pl.ds((9 + k) * N_PAD + base, NODES_PER_W)], p1[k])

    def group(g, _):
        s = pl.ds(g * 16, 16)

        def ld(k):
            return p0[k][s] + p1[k][s]

        a0 = ld(0) + EPS
        a1 = ld(1)
        a2 = ld(2)
        a3 = ld(3) + EPS
        a4 = ld(4)
        a5 = ld(5) + EPS
        b0 = ld(6)
        b1 = ld(7)
        b2 = ld(8)
        c00 = a3 * a5 - a4 * a4
        c01 = a2 * a4 - a1 * a5
        c02 = a1 * a4 - a3 * a2
        c11 = a0 * a5 - a2 * a2
        c12 = a1 * a2 - a0 * a4
        c22 = a0 * a3 - a1 * a1
        inv = 1.0 / (a0 * c00 + a1 * c01 + a2 * c02)
        gbuf[0][s] = (c00 * b0 + c01 * b1 + c02 * b2) * inv
        gbuf[1][s] = (c01 * b0 + c11 * b1 + c12 * b2) * inv
        gbuf[2][s] = (c02 * b0 + c12 * b1 + c22 * b2) * inv
        return _

    lax.fori_loop(0, NODES_PER_W // 16, group, None)
    for k in range(3):
        pltpu.sync_copy(gbuf[k], grad_hbm.at[pl.ds(k * N_PAD + base, NODES_PER_W)])


_accumulate = pl.kernel(
    _accumulate_body,
    mesh=_mesh,
    out_type=jax.ShapeDtypeStruct((NC * 9 * N_PAD,), jnp.float32),
    scratch_types=(
        [pltpu.VMEM((LANES,), jnp.int32) for _ in range(2 * K)]
        + [pltpu.VMEM((CL,), jnp.float32) for _ in range(8)]
        + [pltpu.VMEM((CL,), jnp.float32) for _ in range(9)]
        + [pltpu.VMEM_SHARED((N_PAD,), jnp.float32) for _ in range(9)]
        + [pltpu.VMEM_SHARED((N_PAD,), jnp.float32) for _ in range(4)]
        + [pltpu.SemaphoreType.DMA, pltpu.SemaphoreType.DMA]
    ),
)

_solve = pl.kernel(
    _solve_body,
    mesh=_mesh,
    out_type=jax.ShapeDtypeStruct((3 * N_PAD,), jnp.float32),
    scratch_types=(
        [pltpu.VMEM((NODES_PER_W,), jnp.float32) for _ in range(18)]
        + [pltpu.VMEM((NODES_PER_W,), jnp.float32) for _ in range(3)]
    ),
)


def kernel(pos, phi, edge_index):
    ei = edge_index.astype(jnp.int32)
    pad = jnp.zeros((E_PAD - E,), jnp.int32)
    r_flat = jnp.concatenate([ei[0], pad])
    c_flat = jnp.concatenate([ei[1], pad])
    zeros = jnp.zeros((N_PAD,), jnp.float32)
    npad = jnp.zeros((N_PAD - N,), jnp.float32)
    partials = _accumulate(
        r_flat, c_flat,
        jnp.concatenate([pos[:, 0], npad]), jnp.concatenate([pos[:, 1], npad]),
        jnp.concatenate([pos[:, 2], npad]), jnp.concatenate([phi, npad]), zeros)
    grad = _solve(partials)
    return grad.reshape(3, N_PAD)[:, :N].T


# speedup vs baseline: 674.2512x; 1.1462x over previous
import jax
import jax.numpy as jnp
from jax import lax
from jax.experimental import pallas as pl
from jax.experimental.pallas import tpu as pltpu
from jax.experimental.pallas import tpu_sc as plsc

N = 100000
E = 1600000
EPS = 1e-8

NC = 2
NS = 16
NW = NC * NS
LANES = 128
K = 8
CL = K * LANES

N_PAD = 100352
E_PAD = 1638400
ROWS_PER_W = E_PAD // NW // LANES
CHUNKS = ROWS_PER_W // K
NODES_PER_W = N_PAD // NW
NODES_PER_TILE = N_PAD // NS

IDX_BYTES = 2 * K * LANES * 4
GATHER_BYTES = 2 * 4 * K * LANES * 4
SCATTER_BYTES = 2 * 9 * K * LANES * 4

_mesh = plsc.VectorSubcoreMesh(core_axis_name="c", subcore_axis_name="s")


def _accumulate_body(r_hbm, c_hbm, px_hbm, py_hbm, pz_hbm, ph_hbm, zeros_hbm,
                     out_hbm, *refs):
    idx = (refs[0:16], refs[16:32])
    ebuf = (refs[32:40], refs[40:48])
    vals = (refs[48:57], refs[57:66])
    accs = refs[66:75]
    tbls = refs[75:79]
    gsem = (refs[79], refs[80])
    ssem = (refs[81], refs[82])

    cid = lax.axis_index("c")
    sid = lax.axis_index("s")
    wid = sid * NC + cid

    zsl = pl.ds(sid * NODES_PER_TILE, NODES_PER_TILE)
    for k in range(9):
        pltpu.sync_copy(zeros_hbm.at[zsl], accs[k].at[zsl])
    planes_hbm = (px_hbm, py_hbm, pz_hbm, ph_hbm)
    for comp in range(4):
        pltpu.sync_copy(planes_hbm[comp].at[zsl], tbls[comp].at[zsl])
    plsc.subcore_barrier()

    def stage_and_gather(chunk_id, b):
        base = wid * ROWS_PER_W + chunk_id * K
        cps = []
        for j in range(K):
            cps.append(pltpu.async_copy(
                r_hbm.at[pl.ds((base + j) * LANES, LANES)], idx[b][j], gsem[b]))
            cps.append(pltpu.async_copy(
                c_hbm.at[pl.ds((base + j) * LANES, LANES)], idx[b][K + j],
                gsem[b]))
        for cp in cps:
            cp.wait()
        cps = []
        for j in range(K):
            dsl = pl.ds(j * LANES, LANES)
            for comp in range(4):
                cps.append(pltpu.async_copy(
                    tbls[comp].at[idx[b][j]], ebuf[b][comp].at[dsl], gsem[b]))
                cps.append(pltpu.async_copy(
                    tbls[comp].at[idx[b][K + j]], ebuf[b][4 + comp].at[dsl],
                    gsem[b]))
        return cps

    def compute(b):
        rbuf = ebuf[b][0:4]
        cbuf = ebuf[b][4:8]
        v = vals[b]

        def group(gg, _):
            s = pl.ds(gg * 16, 16)
            dx0 = cbuf[0][s] - rbuf[0][s]
            dx1 = cbuf[1][s] - rbuf[1][s]
            dx2 = cbuf[2][s] - rbuf[2][s]
            dphi = cbuf[3][s] - rbuf[3][s]
            r2 = dx0 * dx0 + dx1 * dx1 + dx2 * dx2
            w = 1.0 / (r2 + EPS * EPS)
            wdx0 = w * dx0
            wdx1 = w * dx1
            wdx2 = w * dx2
            wdphi = w * dphi
            v[0][s] = wdx0 * dx0
            v[1][s] = wdx0 * dx1
            v[2][s] = wdx0 * dx2
            v[3][s] = wdx1 * dx1
            v[4][s] = wdx1 * dx2
            v[5][s] = wdx2 * dx2
            v[6][s] = wdphi * dx0
            v[7][s] = wdphi * dx1
            v[8][s] = wdphi * dx2
            return _

        lax.fori_loop(0, CL // 16, group, None)

    def scatter_descs(b, fire):
        op = pltpu.async_copy if fire else (
            lambda s, d, sem, add=False: pltpu.make_async_copy(s, d, sem))
        cps = []
        for j in range(K):
            dsl = pl.ds(j * LANES, LANES)
            for k in range(9):
                cps.append(op(
                    vals[b][k].at[dsl], accs[k].at[idx[b][j]], ssem[b],
                    add=True))
                cps.append(op(
                    vals[b][k].at[dsl], accs[k].at[idx[b][K + j]], ssem[b],
                    add=True))
        return cps

    def drain_scatters(b):
        for cp in scatter_descs(b, fire=False):
            cp.wait()

    def chunk_pair(i, _):
        gcps = []
        for b in range(2):
            @pl.when(i > 0)
            def _drain(b=b):
                drain_scatters(b)
            gcps.append(stage_and_gather(2 * i + b, b))
        for b in range(2):
            for cp in gcps[b]:
                cp.wait()
            compute(b)
            scatter_descs(b, fire=True)
        return _

    lax.fori_loop(0, CHUNKS // 2, chunk_pair, None)
    drain_scatters(0)
    drain_scatters(1)

    plsc.subcore_barrier()
    for k in range(9):
        pltpu.sync_copy(
            accs[k].at[zsl],
            out_hbm.at[pl.ds((cid * 9 + k) * N_PAD + sid * NODES_PER_TILE,
                             NODES_PER_TILE)])


def _solve_body(part_hbm, grad_hbm, *refs):
    p0 = refs[0:9]
    p1 = refs[9:18]
    gbuf = refs[18:21]

    cid = lax.axis_index("c")
    sid = lax.axis_index("s")
    wid = sid * NC + cid
    base = wid * NODES_PER_W

    for k in range(9):
        pltpu.sync_copy(part_hbm.at[pl.ds(k * N_PAD + base, NODES_PER_W)], p0[k])
        pltpu.sync_copy(part_hbm.at[pl.ds((9 + k) * N_PAD + base, NODES_PER_W)], p1[k])

    def group(g, _):
        s = pl.ds(g * 16, 16)

        def ld(k):
            return p0[k][s] + p1[k][s]

        a0 = ld(0) + EPS
        a1 = ld(1)
        a2 = ld(2)
        a3 = ld(3) + EPS
        a4 = ld(4)
        a5 = ld(5) + EPS
        b0 = ld(6)
        b1 = ld(7)
        b2 = ld(8)
        c00 = a3 * a5 - a4 * a4
        c01 = a2 * a4 - a1 * a5
        c02 = a1 * a4 - a3 * a2
        c11 = a0 * a5 - a2 * a2
        c12 = a1 * a2 - a0 * a4
        c22 = a0 * a3 - a1 * a1
        inv = 1.0 / (a0 * c00 + a1 * c01 + a2 * c02)
        gbuf[0][s] = (c00 * b0 + c01 * b1 + c02 * b2) * inv
        gbuf[1][s] = (c01 * b0 + c11 * b1 + c12 * b2) * inv
        gbuf[2][s] = (c02 * b0 + c12 * b1 + c22 * b2) * inv
        return _

    lax.fori_loop(0, NODES_PER_W // 16, group, None)
    for k in range(3):
        pltpu.sync_copy(gbuf[k], grad_hbm.at[pl.ds(k * N_PAD + base, NODES_PER_W)])


_accumulate = pl.kernel(
    _accumulate_body,
    mesh=_mesh,
    out_type=jax.ShapeDtypeStruct((NC * 9 * N_PAD,), jnp.float32),
    scratch_types=(
        [pltpu.VMEM((LANES,), jnp.int32) for _ in range(32)]
        + [pltpu.VMEM((CL,), jnp.float32) for _ in range(16)]
        + [pltpu.VMEM((CL,), jnp.float32) for _ in range(18)]
        + [pltpu.VMEM_SHARED((N_PAD,), jnp.float32) for _ in range(9)]
        + [pltpu.VMEM_SHARED((N_PAD,), jnp.float32) for _ in range(4)]
        + [pltpu.SemaphoreType.DMA for _ in range(4)]
    ),
)

_solve = pl.kernel(
    _solve_body,
    mesh=_mesh,
    out_type=jax.ShapeDtypeStruct((3 * N_PAD,), jnp.float32),
    scratch_types=(
        [pltpu.VMEM((NODES_PER_W,), jnp.float32) for _ in range(18)]
        + [pltpu.VMEM((NODES_PER_W,), jnp.float32) for _ in range(3)]
    ),
)


def kernel(pos, phi, edge_index):
    ei = edge_index.astype(jnp.int32)
    pad = N + jnp.arange(E_PAD - E, dtype=jnp.int32) % (N_PAD - N)
    r_flat = jnp.concatenate([ei[0], pad])
    c_flat = jnp.concatenate([ei[1], pad])
    zeros = jnp.zeros((N_PAD,), jnp.float32)
    npad = jnp.zeros((N_PAD - N,), jnp.float32)
    partials = _accumulate(
        r_flat, c_flat,
        jnp.concatenate([pos[:, 0], npad]), jnp.concatenate([pos[:, 1], npad]),
        jnp.concatenate([pos[:, 2], npad]), jnp.concatenate([phi, npad]), zeros)
    grad = _solve(partials)
    return grad.reshape(3, N_PAD)[:, :N].T


# speedup vs baseline: 678.4908x; 1.0063x over previous
import jax
import jax.numpy as jnp
from jax import lax
from jax.experimental import pallas as pl
from jax.experimental.pallas import tpu as pltpu
from jax.experimental.pallas import tpu_sc as plsc

N = 100000
E = 1600000
EPS = 1e-8

NC = 2
NS = 16
NW = NC * NS
LANES = 128
K = 10
CL = K * LANES

N_PAD = 100352
E_PAD = 1638400
ROWS_PER_W = E_PAD // NW // LANES
CHUNKS = ROWS_PER_W // K
assert ROWS_PER_W % K == 0 and CHUNKS % 2 == 0
NODES_PER_W = N_PAD // NW
NODES_PER_TILE = N_PAD // NS

IDX_BYTES = 2 * K * LANES * 4
GATHER_BYTES = 2 * 4 * K * LANES * 4
SCATTER_BYTES = 2 * 9 * K * LANES * 4

_mesh = plsc.VectorSubcoreMesh(core_axis_name="c", subcore_axis_name="s")


def _accumulate_body(r_hbm, c_hbm, px_hbm, py_hbm, pz_hbm, ph_hbm, zeros_hbm,
                     out_hbm, *refs):
    o = 4 * K
    idx = (refs[0:2 * K], refs[2 * K:4 * K])
    ebuf = (refs[o:o + 8], refs[o + 8:o + 16])
    vals = (refs[o + 16:o + 25], refs[o + 25:o + 34])
    accs = refs[o + 34:o + 43]
    tbls = refs[o + 43:o + 47]
    gsem = (refs[o + 47], refs[o + 48])
    ssem = (refs[o + 49], refs[o + 50])

    cid = lax.axis_index("c")
    sid = lax.axis_index("s")
    wid = sid * NC + cid

    zsl = pl.ds(sid * NODES_PER_TILE, NODES_PER_TILE)
    for k in range(9):
        pltpu.sync_copy(zeros_hbm.at[zsl], accs[k].at[zsl])
    planes_hbm = (px_hbm, py_hbm, pz_hbm, ph_hbm)
    for comp in range(4):
        pltpu.sync_copy(planes_hbm[comp].at[zsl], tbls[comp].at[zsl])
    plsc.subcore_barrier()

    def stage_and_gather(chunk_id, b):
        base = wid * ROWS_PER_W + chunk_id * K
        cps = []
        for j in range(K):
            cps.append(pltpu.async_copy(
                r_hbm.at[pl.ds((base + j) * LANES, LANES)], idx[b][j], gsem[b]))
            cps.append(pltpu.async_copy(
                c_hbm.at[pl.ds((base + j) * LANES, LANES)], idx[b][K + j],
                gsem[b]))
        for cp in cps:
            cp.wait()
        cps = []
        for j in range(K):
            dsl = pl.ds(j * LANES, LANES)
            for comp in range(4):
                cps.append(pltpu.async_copy(
                    tbls[comp].at[idx[b][j]], ebuf[b][comp].at[dsl], gsem[b]))
                cps.append(pltpu.async_copy(
                    tbls[comp].at[idx[b][K + j]], ebuf[b][4 + comp].at[dsl],
                    gsem[b]))
        return cps

    def compute(b):
        rbuf = ebuf[b][0:4]
        cbuf = ebuf[b][4:8]
        v = vals[b]

        def group(gg, _):
            s = pl.ds(gg * 16, 16)
            dx0 = cbuf[0][s] - rbuf[0][s]
            dx1 = cbuf[1][s] - rbuf[1][s]
            dx2 = cbuf[2][s] - rbuf[2][s]
            dphi = cbuf[3][s] - rbuf[3][s]
            r2 = dx0 * dx0 + dx1 * dx1 + dx2 * dx2
            w = 1.0 / (r2 + EPS * EPS)
            wdx0 = w * dx0
            wdx1 = w * dx1
            wdx2 = w * dx2
            wdphi = w * dphi
            v[0][s] = wdx0 * dx0
            v[1][s] = wdx0 * dx1
            v[2][s] = wdx0 * dx2
            v[3][s] = wdx1 * dx1
            v[4][s] = wdx1 * dx2
            v[5][s] = wdx2 * dx2
            v[6][s] = wdphi * dx0
            v[7][s] = wdphi * dx1
            v[8][s] = wdphi * dx2
            return _

        lax.fori_loop(0, CL // 16, group, None)

    def scatter_descs(b, fire):
        op = pltpu.async_copy if fire else (
            lambda s, d, sem, add=False: pltpu.make_async_copy(s, d, sem))
        cps = []
        for j in range(K):
            dsl = pl.ds(j * LANES, LANES)
            for k in range(9):
                cps.append(op(
                    vals[b][k].at[dsl], accs[k].at[idx[b][j]], ssem[b],
                    add=True))
                cps.append(op(
                    vals[b][k].at[dsl], accs[k].at[idx[b][K + j]], ssem[b],
                    add=True))
        return cps

    def drain_scatters(b):
        for cp in scatter_descs(b, fire=False):
            cp.wait()

    def chunk_pair(i, _):
        gcps = []
        for b in range(2):
            @pl.when(i > 0)
            def _drain(b=b):
                drain_scatters(b)
            gcps.append(stage_and_gather(2 * i + b, b))
        for b in range(2):
            for cp in gcps[b]:
                cp.wait()
            compute(b)
            scatter_descs(b, fire=True)
        return _

    lax.fori_loop(0, CHUNKS // 2, chunk_pair, None)
    drain_scatters(0)
    drain_scatters(1)

    plsc.subcore_barrier()
    for k in range(9):
        pltpu.sync_copy(
            accs[k].at[zsl],
            out_hbm.at[pl.ds((cid * 9 + k) * N_PAD + sid * NODES_PER_TILE,
                             NODES_PER_TILE)])


def _solve_body(part_hbm, grad_hbm, *refs):
    p0 = refs[0:9]
    p1 = refs[9:18]
    gbuf = refs[18:21]

    cid = lax.axis_index("c")
    sid = lax.axis_index("s")
    wid = sid * NC + cid
    base = wid * NODES_PER_W

    for k in range(9):
        pltpu.sync_copy(part_hbm.at[pl.ds(k * N_PAD + base, NODES_PER_W)], p0[k])
        pltpu.sync_copy(part_hbm.at[pl.ds((9 + k) * N_PAD + base, NODES_PER_W)], p1[k])

    def group(g, _):
        s = pl.ds(g * 16, 16)

        def ld(k):
            return p0[k][s] + p1[k][s]

        a0 = ld(0) + EPS
        a1 = ld(1)
        a2 = ld(2)
        a3 = ld(3) + EPS
        a4 = ld(4)
        a5 = ld(5) + EPS
        b0 = ld(6)
        b1 = ld(7)
        b2 = ld(8)
        c00 = a3 * a5 - a4 * a4
        c01 = a2 * a4 - a1 * a5
        c02 = a1 * a4 - a3 * a2
        c11 = a0 * a5 - a2 * a2
        c12 = a1 * a2 - a0 * a4
        c22 = a0 * a3 - a1 * a1
        inv = 1.0 / (a0 * c00 + a1 * c01 + a2 * c02)
        gbuf[0][s] = (c00 * b0 + c01 * b1 + c02 * b2) * inv
        gbuf[1][s] = (c01 * b0 + c11 * b1 + c12 * b2) * inv
        gbuf[2][s] = (c02 * b0 + c12 * b1 + c22 * b2) * inv
        return _

    lax.fori_loop(0, NODES_PER_W // 16, group, None)
    for k in range(3):
        pltpu.sync_copy(gbuf[k], grad_hbm.at[pl.ds(k * N_PAD + base, NODES_PER_W)])


_accumulate = pl.kernel(
    _accumulate_body,
    mesh=_mesh,
    out_type=jax.ShapeDtypeStruct((NC * 9 * N_PAD,), jnp.float32),
    scratch_types=(
        [pltpu.VMEM((LANES,), jnp.int32) for _ in range(4 * K)]
        + [pltpu.VMEM((CL,), jnp.float32) for _ in range(16)]
        + [pltpu.VMEM((CL,), jnp.float32) for _ in range(18)]
        + [pltpu.VMEM_SHARED((N_PAD,), jnp.float32) for _ in range(9)]
        + [pltpu.VMEM_SHARED((N_PAD,), jnp.float32) for _ in range(4)]
        + [pltpu.SemaphoreType.DMA for _ in range(4)]
    ),
)

_solve = pl.kernel(
    _solve_body,
    mesh=_mesh,
    out_type=jax.ShapeDtypeStruct((3 * N_PAD,), jnp.float32),
    scratch_types=(
        [pltpu.VMEM((NODES_PER_W,), jnp.float32) for _ in range(18)]
        + [pltpu.VMEM((NODES_PER_W,), jnp.float32) for _ in range(3)]
    ),
)


def kernel(pos, phi, edge_index):
    ei = edge_index.astype(jnp.int32)
    pad = N + jnp.arange(E_PAD - E, dtype=jnp.int32) % (N_PAD - N)
    r_flat = jnp.concatenate([ei[0], pad])
    c_flat = jnp.concatenate([ei[1], pad])
    zeros = jnp.zeros((N_PAD,), jnp.float32)
    npad = jnp.zeros((N_PAD - N,), jnp.float32)
    partials = _accumulate(
        r_flat, c_flat,
        jnp.concatenate([pos[:, 0], npad]), jnp.concatenate([pos[:, 1], npad]),
        jnp.concatenate([pos[:, 2], npad]), jnp.concatenate([phi, npad]), zeros)
    grad = _solve(partials)
    return grad.reshape(3, N_PAD)[:, :N].T
